# baseline (speedup 1.0000x reference)
import jax
import jax.numpy as jnp
from jax.experimental import pallas as pl
from jax.experimental.pallas import tpu as pltpu

_NUM_SNPS = 100000
_NUM_IMPACTS = 16
_EMB = 16
_BATCH = 1024

_COLS = 1280
_GRID = -(-_NUM_SNPS // _COLS)
_PADDED = _GRID * _COLS


def _head_body(emb_ref, wpt_ref, bp_ref, gamma_ref, beta_ref, wsb_ref,
               bsbb_ref, tab_ref):
    h = jnp.dot(emb_ref[...], wpt_ref[...],
                preferred_element_type=jnp.float32) + bp_ref[...]
    mu = jnp.mean(h, axis=-1, keepdims=True)
    var = jnp.mean((h - mu) ** 2, axis=-1, keepdims=True)
    h = (h - mu) / jnp.sqrt(var + 1e-5) * gamma_ref[...] + beta_ref[...]
    h = jnp.maximum(h, 0.0)
    tab_ref[...] = jnp.dot(h, wsb_ref[...],
                           preferred_element_type=jnp.float32) + bsbb_ref[...]


def _dense_body(tab_ref, idx_ref, x_ref, o_ref):
    idx = idx_ref[0]
    s = jnp.full(idx.shape, tab_ref[0, 0], jnp.float32)
    b = jnp.full(idx.shape, tab_ref[0, 1], jnp.float32)
    for k in range(1, _NUM_IMPACTS):
        m = idx == k
        s = jnp.where(m, tab_ref[k, 0], s)
        b = jnp.where(m, tab_ref[k, 1], b)
    xx = x_ref[...]
    o_ref[...] = xx * (2.0 / (1.0 + jnp.exp(-(xx * s + b))))


def kernel(x, impact_indices, emb, Wp, bp, gamma, beta, ws, bs, wb, bb):
    wpt = Wp.T
    wsb = jnp.concatenate([ws, wb], axis=1)
    bsbb = jnp.concatenate([bs, bb]).reshape(1, 2)

    tab = pl.pallas_call(
        _head_body,
        out_shape=jax.ShapeDtypeStruct((_NUM_IMPACTS, 2), jnp.float32),
    )(emb, wpt, bp.reshape(1, _EMB), gamma.reshape(1, _EMB),
      beta.reshape(1, _EMB), wsb, bsbb)

    idx = jnp.pad(impact_indices, (0, _PADDED - _NUM_SNPS))
    idx = idx.reshape(_GRID, 1, _COLS)

    out = pl.pallas_call(
        _dense_body,
        grid=(_GRID,),
        in_specs=[
            pl.BlockSpec(memory_space=pltpu.SMEM),
            pl.BlockSpec((1, 1, _COLS), lambda j: (j, 0, 0)),
            pl.BlockSpec((_BATCH, _COLS), lambda j: (0, j)),
        ],
        out_specs=pl.BlockSpec((_BATCH, _COLS), lambda j: (0, j)),
        out_shape=jax.ShapeDtypeStruct((_BATCH, _NUM_SNPS), jnp.float32),
    )(tab, idx, x)
    return out


# speedup vs baseline: 1.0019x; 1.0019x over previous
import jax
import jax.numpy as jnp
from jax.experimental import pallas as pl
from jax.experimental.pallas import tpu as pltpu

_NUM_SNPS = 100000
_NUM_IMPACTS = 16
_EMB = 16
_BATCH = 1024

_COLS = 1280
_GRID = -(-_NUM_SNPS // _COLS)
_PADDED = _GRID * _COLS


def _head_body(emb_ref, wpt_ref, bp_ref, gamma_ref, beta_ref, wsb_ref,
               bsbb_ref, tab_ref):
    h = jnp.dot(emb_ref[...], wpt_ref[...],
                preferred_element_type=jnp.float32) + bp_ref[...]
    mu = jnp.mean(h, axis=-1, keepdims=True)
    var = jnp.mean((h - mu) ** 2, axis=-1, keepdims=True)
    h = (h - mu) / jnp.sqrt(var + 1e-5) * gamma_ref[...] + beta_ref[...]
    h = jnp.maximum(h, 0.0)
    tab_ref[...] = jnp.dot(h, wsb_ref[...],
                           preferred_element_type=jnp.float32) + bsbb_ref[...]


def _dense_body(tab_ref, idx_ref, x_ref, o_ref):
    idx = idx_ref[0]
    s = jnp.full(idx.shape, tab_ref[0, 0], jnp.float32)
    b = jnp.full(idx.shape, tab_ref[0, 1], jnp.float32)
    for k in range(1, _NUM_IMPACTS):
        m = idx == k
        s = jnp.where(m, tab_ref[k, 0], s)
        b = jnp.where(m, tab_ref[k, 1], b)
    xx = x_ref[...]
    o_ref[...] = xx + xx * jnp.tanh(xx * (0.5 * s) + 0.5 * b)


def kernel(x, impact_indices, emb, Wp, bp, gamma, beta, ws, bs, wb, bb):
    wpt = Wp.T
    wsb = jnp.concatenate([ws, wb], axis=1)
    bsbb = jnp.concatenate([bs, bb]).reshape(1, 2)

    tab = pl.pallas_call(
        _head_body,
        out_shape=jax.ShapeDtypeStruct((_NUM_IMPACTS, 2), jnp.float32),
    )(emb, wpt, bp.reshape(1, _EMB), gamma.reshape(1, _EMB),
      beta.reshape(1, _EMB), wsb, bsbb)

    idx = jnp.pad(impact_indices, (0, _PADDED - _NUM_SNPS))
    idx = idx.reshape(_GRID, 1, _COLS)

    out = pl.pallas_call(
        _dense_body,
        grid=(_GRID,),
        in_specs=[
            pl.BlockSpec(memory_space=pltpu.SMEM),
            pl.BlockSpec((1, 1, _COLS), lambda j: (j, 0, 0)),
            pl.BlockSpec((_BATCH, _COLS), lambda j: (0, j)),
        ],
        out_specs=pl.BlockSpec((_BATCH, _COLS), lambda j: (0, j)),
        out_shape=jax.ShapeDtypeStruct((_BATCH, _NUM_SNPS), jnp.float32),
    )(tab, idx, x)
    return out


# speedup vs baseline: 1.0029x; 1.0010x over previous
import jax
import jax.numpy as jnp
from jax.experimental import pallas as pl
from jax.experimental.pallas import tpu as pltpu

_NUM_SNPS = 100000
_NUM_IMPACTS = 16
_EMB = 16
_BATCH = 1024

_COLS = 1280
_GRID = -(-_NUM_SNPS // _COLS)
_PADDED = _GRID * _COLS


def _head_body(emb_ref, wpt_ref, bp_ref, gamma_ref, beta_ref, wsb_ref,
               bsbb_ref, tab_ref):
    h = jnp.dot(emb_ref[...], wpt_ref[...],
                preferred_element_type=jnp.float32) + bp_ref[...]
    mu = jnp.mean(h, axis=-1, keepdims=True)
    var = jnp.mean((h - mu) ** 2, axis=-1, keepdims=True)
    h = (h - mu) / jnp.sqrt(var + 1e-5) * gamma_ref[...] + beta_ref[...]
    h = jnp.maximum(h, 0.0)
    tab_ref[...] = jnp.dot(h, wsb_ref[...],
                           preferred_element_type=jnp.float32) + bsbb_ref[...]


def _dense_body(tab_ref, idx_ref, x_ref, o_ref):
    idx = idx_ref[0]
    s = jnp.full(idx.shape, tab_ref[0, 0], jnp.float32)
    b = jnp.full(idx.shape, tab_ref[0, 1], jnp.float32)
    for k in range(1, _NUM_IMPACTS):
        m = idx == k
        s = jnp.where(m, tab_ref[k, 0], s)
        b = jnp.where(m, tab_ref[k, 1], b)
    xx = x_ref[...]
    o_ref[...] = xx + xx * jnp.tanh(xx * (0.5 * s) + 0.5 * b)


def kernel(x, impact_indices, emb, Wp, bp, gamma, beta, ws, bs, wb, bb):
    wpt = Wp.T
    wsb = jnp.concatenate([ws, wb], axis=1)
    bsbb = jnp.concatenate([bs, bb]).reshape(1, 2)

    tab = pl.pallas_call(
        _head_body,
        out_shape=jax.ShapeDtypeStruct((_NUM_IMPACTS, 2), jnp.float32),
    )(emb, wpt, bp.reshape(1, _EMB), gamma.reshape(1, _EMB),
      beta.reshape(1, _EMB), wsb, bsbb)

    idx = jnp.pad(impact_indices, (0, _PADDED - _NUM_SNPS))
    idx = idx.reshape(_GRID, 1, _COLS)

    out = pl.pallas_call(
        _dense_body,
        grid=(_GRID,),
        in_specs=[
            pl.BlockSpec(memory_space=pltpu.SMEM),
            pl.BlockSpec((1, 1, _COLS), lambda j: (j, 0, 0)),
            pl.BlockSpec((_BATCH, _COLS), lambda j: (0, j)),
        ],
        out_specs=pl.BlockSpec((_BATCH, _COLS), lambda j: (0, j)),
        out_shape=jax.ShapeDtypeStruct((_BATCH, _NUM_SNPS), jnp.float32),
        compiler_params=pltpu.CompilerParams(
            dimension_semantics=("parallel",)),
    )(tab, idx, x)
    return out


# speedup vs baseline: 1.0048x; 1.0019x over previous
import jax
import jax.numpy as jnp
from jax.experimental import pallas as pl
from jax.experimental.pallas import tpu as pltpu

_NUM_SNPS = 100000
_NUM_IMPACTS = 16
_EMB = 16
_BATCH = 1024

_COLS = 2560
_GRID = -(-_NUM_SNPS // _COLS)
_PADDED = _GRID * _COLS


def _head_body(emb_ref, wpt_ref, bp_ref, gamma_ref, beta_ref, wsb_ref,
               bsbb_ref, tab_ref):
    h = jnp.dot(emb_ref[...], wpt_ref[...],
                preferred_element_type=jnp.float32) + bp_ref[...]
    mu = jnp.mean(h, axis=-1, keepdims=True)
    var = jnp.mean((h - mu) ** 2, axis=-1, keepdims=True)
    h = (h - mu) / jnp.sqrt(var + 1e-5) * gamma_ref[...] + beta_ref[...]
    h = jnp.maximum(h, 0.0)
    tab_ref[...] = jnp.dot(h, wsb_ref[...],
                           preferred_element_type=jnp.float32) + bsbb_ref[...]


def _dense_body(tab_ref, idx_ref, x_ref, o_ref):
    idx = idx_ref[0]
    s = jnp.full(idx.shape, tab_ref[0, 0], jnp.float32)
    b = jnp.full(idx.shape, tab_ref[0, 1], jnp.float32)
    for k in range(1, _NUM_IMPACTS):
        m = idx == k
        s = jnp.where(m, tab_ref[k, 0], s)
        b = jnp.where(m, tab_ref[k, 1], b)
    xx = x_ref[...]
    o_ref[...] = xx + xx * jnp.tanh(xx * (0.5 * s) + 0.5 * b)


def kernel(x, impact_indices, emb, Wp, bp, gamma, beta, ws, bs, wb, bb):
    wpt = Wp.T
    wsb = jnp.concatenate([ws, wb], axis=1)
    bsbb = jnp.concatenate([bs, bb]).reshape(1, 2)

    tab = pl.pallas_call(
        _head_body,
        out_shape=jax.ShapeDtypeStruct((_NUM_IMPACTS, 2), jnp.float32),
    )(emb, wpt, bp.reshape(1, _EMB), gamma.reshape(1, _EMB),
      beta.reshape(1, _EMB), wsb, bsbb)

    idx = jnp.pad(impact_indices, (0, _PADDED - _NUM_SNPS))
    idx = idx.reshape(_GRID, 1, _COLS)

    out = pl.pallas_call(
        _dense_body,
        grid=(_GRID,),
        in_specs=[
            pl.BlockSpec(memory_space=pltpu.SMEM),
            pl.BlockSpec((1, 1, _COLS), lambda j: (j, 0, 0)),
            pl.BlockSpec((_BATCH, _COLS), lambda j: (0, j)),
        ],
        out_specs=pl.BlockSpec((_BATCH, _COLS), lambda j: (0, j)),
        out_shape=jax.ShapeDtypeStruct((_BATCH, _NUM_SNPS), jnp.float32),
        compiler_params=pltpu.CompilerParams(
            dimension_semantics=("parallel",)),
    )(tab, idx, x)
    return out
